# baseline (speedup 1.0000x reference)
import functools

import jax
import jax.numpy as jnp
from jax import lax
from jax.experimental import pallas as pl
from jax.experimental.pallas import tpu as pltpu
from jax.experimental.pallas import tpu_sc as plsc

_VOCAB = 100000
_EMB = 128
_DP = 16
_BATCH = 4096
_SEQ = 200
_NW = 32
_RPW = _BATCH // _NW
_C = 8
_NCHUNK = _RPW // _C

_MM_BLK = 16384


def _matmul_body(emb3_ref, wp_ref, out_ref):
    wp = wp_ref[...]
    parts = []
    for s in range(8):
        parts.append(
            jnp.dot(
                emb3_ref[:, s, :],
                wp,
                preferred_element_type=jnp.float32,
            )
        )
    out_ref[...] = jnp.concatenate(parts, axis=-1)


def _head_table(emb, wp):
    emb3 = emb.reshape(_VOCAB // 8, 8, _EMB)
    blk = _MM_BLK // 8
    return pl.pallas_call(
        _matmul_body,
        grid=(pl.cdiv(_VOCAB // 8, blk),),
        in_specs=[
            pl.BlockSpec((blk, 8, _EMB), lambda i: (i, 0, 0)),
            pl.BlockSpec((_EMB, _DP), lambda i: (0, 0)),
        ],
        out_specs=pl.BlockSpec((blk, _EMB), lambda i: (i, 0)),
        out_shape=jax.ShapeDtypeStruct((_VOCAB // 8, _EMB), jnp.float32),
    )(emb3, wp)


def _pooled_head(table, ids_flat, bpad):
    mesh = plsc.VectorSubcoreMesh(core_axis_name="c", subcore_axis_name="s")

    cl = _C * _SEQ

    @functools.partial(
        pl.kernel,
        out_type=jax.ShapeDtypeStruct((_BATCH, _DP), jnp.float32),
        mesh=mesh,
        scratch_types=[
            pltpu.VMEM((_RPW * _SEQ,), jnp.int32),
            pltpu.VMEM((2, cl, _DP), jnp.float32),
            pltpu.VMEM((_C, _DP), jnp.float32),
            pltpu.VMEM((_DP,), jnp.float32),
            pltpu.SemaphoreType.DMA((2,)),
        ],
        compiler_params=pltpu.CompilerParams(use_tc_tiling_on_sc=False),
    )
    def pool(table_hbm, ids_hbm, b_hbm, out_hbm, idx_v, rows_v, outc_v, b_v, sem):
        wid = lax.axis_index("s") * 2 + lax.axis_index("c")
        pltpu.sync_copy(b_hbm, b_v)
        b_vec = b_v[...]
        pltpu.sync_copy(ids_hbm.at[pl.ds(wid * _RPW * _SEQ, _RPW * _SEQ)], idx_v)

        def gather_desc(ch, buf):
            return pltpu.make_async_copy(
                table_hbm.at[idx_v.at[pl.ds(ch * cl, cl)]], rows_v.at[buf], sem.at[buf]
            )

        gather_desc(0, 0).start()

        @pl.loop(0, _NCHUNK // 2)
        def _pair(k):
            for buf in range(2):
                ch = 2 * k + buf
                gather_desc(ch, buf).wait()

                @pl.when(ch + 1 < _NCHUNK)
                def _prefetch():
                    gather_desc(ch + 1, 1 - buf).start()

                for c in range(_C):
                    base = c * _SEQ

                    def body(j, acc, base=base, buf=buf):
                        j8 = base + 8 * j
                        r = [rows_v[buf, j8 + t] for t in range(8)]
                        s0 = (r[0] + r[1]) + (r[2] + r[3])
                        s1 = (r[4] + r[5]) + (r[6] + r[7])
                        return acc + (s0 + s1)

                    outc_v[c] = lax.fori_loop(0, _SEQ // 8, body, b_vec)
                pltpu.sync_copy(
                    outc_v, out_hbm.at[pl.ds(wid * _RPW + ch * _C, _C)]
                )

    return pool(table, ids_flat, bpad)


def kernel(ids, emb, W, b):
    wp = jnp.zeros((_EMB, _DP), jnp.float32).at[:, :4].set(W.T * (1.0 / _SEQ))
    bpad = jnp.zeros((_DP,), jnp.float32).at[:4].set(b)
    table = _head_table(emb, wp).reshape(_VOCAB, _DP)
    ids_flat = ids.reshape(-1).astype(jnp.int32)
    out = _pooled_head(table, ids_flat, bpad)
    return out[:, :4]


# speedup vs baseline: 1.1191x; 1.1191x over previous
import functools

import jax
import jax.numpy as jnp
from jax import lax
from jax.experimental import pallas as pl
from jax.experimental.pallas import tpu as pltpu
from jax.experimental.pallas import tpu_sc as plsc

_VOCAB = 100000
_EMB = 128
_DP = 16
_BATCH = 4096
_SEQ = 200
_NW = 32
_RPW = _BATCH // _NW
_C = 8
_NCHUNK = _RPW // _C

_MM_BLK = 8192


def _matmul_body(emb3_ref, wp_ref, out_ref):
    wp = wp_ref[...]
    parts = []
    for s in range(8):
        parts.append(
            jnp.dot(
                emb3_ref[:, s, :],
                wp,
                preferred_element_type=jnp.float32,
            )
        )
    out_ref[...] = jnp.concatenate(parts, axis=-1)


def _head_table(emb, wp):
    emb3 = emb.reshape(_VOCAB // 8, 8, _EMB)
    blk = _MM_BLK // 8
    return pl.pallas_call(
        _matmul_body,
        grid=(pl.cdiv(_VOCAB // 8, blk),),
        in_specs=[
            pl.BlockSpec((blk, 8, _EMB), lambda i: (i, 0, 0)),
            pl.BlockSpec((_EMB, _DP), lambda i: (0, 0)),
        ],
        out_specs=pl.BlockSpec((blk, _EMB), lambda i: (i, 0)),
        out_shape=jax.ShapeDtypeStruct((_VOCAB // 8, _EMB), jnp.float32),
    )(emb3, wp)


def _pooled_head(table, ids_flat, bpad):
    mesh = plsc.VectorSubcoreMesh(core_axis_name="c", subcore_axis_name="s")

    cl = _C * _SEQ

    @functools.partial(
        pl.kernel,
        out_type=jax.ShapeDtypeStruct((_BATCH, _DP), jnp.float32),
        mesh=mesh,
        scratch_types=[
            pltpu.VMEM((_RPW * _SEQ,), jnp.int32),
            pltpu.VMEM((4, cl, _DP), jnp.float32),
            pltpu.VMEM((_C, _DP), jnp.float32),
            pltpu.VMEM((_DP,), jnp.float32),
            pltpu.SemaphoreType.DMA((4,)),
        ],
        compiler_params=pltpu.CompilerParams(use_tc_tiling_on_sc=False),
    )
    def pool(table_hbm, ids_hbm, b_hbm, out_hbm, idx_v, rows_v, outc_v, b_v, sem):
        wid = lax.axis_index("s") * 2 + lax.axis_index("c")
        pltpu.sync_copy(b_hbm, b_v)
        b_vec = b_v[...]
        pltpu.sync_copy(ids_hbm.at[pl.ds(wid * _RPW * _SEQ, _RPW * _SEQ)], idx_v)

        def gather_desc(ch, buf):
            return pltpu.make_async_copy(
                table_hbm.at[idx_v.at[pl.ds(ch * cl, cl)]], rows_v.at[buf], sem.at[buf]
            )

        for p in range(3):
            gather_desc(p, p).start()

        @pl.loop(0, _NCHUNK // 4)
        def _quad(k):
            for buf in range(4):
                ch = 4 * k + buf
                gather_desc(ch, buf).wait()

                @pl.when(ch + 3 < _NCHUNK)
                def _prefetch():
                    gather_desc(ch + 3, (buf + 3) % 4).start()

                for c in range(_C):
                    base = c * _SEQ

                    def body(j, acc, base=base, buf=buf):
                        j8 = base + 8 * j
                        r = [rows_v[buf, j8 + t] for t in range(8)]
                        s0 = (r[0] + r[1]) + (r[2] + r[3])
                        s1 = (r[4] + r[5]) + (r[6] + r[7])
                        return acc + (s0 + s1)

                    outc_v[c] = lax.fori_loop(0, _SEQ // 8, body, b_vec)
                pltpu.sync_copy(
                    outc_v, out_hbm.at[pl.ds(wid * _RPW + ch * _C, _C)]
                )

    return pool(table, ids_flat, bpad)


def kernel(ids, emb, W, b):
    wp = jnp.zeros((_EMB, _DP), jnp.float32).at[:, :4].set(W.T * (1.0 / _SEQ))
    bpad = jnp.zeros((_DP,), jnp.float32).at[:4].set(b)
    table = _head_table(emb, wp).reshape(_VOCAB, _DP)
    ids_flat = ids.reshape(-1).astype(jnp.int32)
    out = _pooled_head(table, ids_flat, bpad)
    return out[:, :4]


# speedup vs baseline: 1.1235x; 1.0040x over previous
import functools

import jax
import jax.numpy as jnp
from jax import lax
from jax.experimental import pallas as pl
from jax.experimental.pallas import tpu as pltpu
from jax.experimental.pallas import tpu_sc as plsc

_VOCAB = 100000
_EMB = 128
_DP = 16
_BATCH = 4096
_SEQ = 200
_NW = 32
_RPW = _BATCH // _NW
_C = 4
_NCHUNK = _RPW // _C

_MM_BLK = 8192


def _matmul_body(emb3_ref, wp_ref, out_ref):
    wp = wp_ref[...]
    parts = []
    for s in range(8):
        parts.append(
            jnp.dot(
                emb3_ref[:, s, :],
                wp,
                preferred_element_type=jnp.float32,
            )
        )
    out_ref[...] = jnp.concatenate(parts, axis=-1)


def _head_table(emb, wp):
    emb3 = emb.reshape(_VOCAB // 8, 8, _EMB)
    blk = _MM_BLK // 8
    return pl.pallas_call(
        _matmul_body,
        grid=(pl.cdiv(_VOCAB // 8, blk),),
        in_specs=[
            pl.BlockSpec((blk, 8, _EMB), lambda i: (i, 0, 0)),
            pl.BlockSpec((_EMB, _DP), lambda i: (0, 0)),
        ],
        out_specs=pl.BlockSpec((blk, _EMB), lambda i: (i, 0)),
        out_shape=jax.ShapeDtypeStruct((_VOCAB // 8, _EMB), jnp.float32),
    )(emb3, wp)


def _pooled_head(table, ids_flat, bpad):
    mesh = plsc.VectorSubcoreMesh(core_axis_name="c", subcore_axis_name="s")

    cl = _C * _SEQ

    @functools.partial(
        pl.kernel,
        out_type=jax.ShapeDtypeStruct((_BATCH, _DP), jnp.float32),
        mesh=mesh,
        scratch_types=[
            pltpu.VMEM((_RPW * _SEQ,), jnp.int32),
            pltpu.VMEM((8, cl, _DP), jnp.float32),
            pltpu.VMEM((_C, _DP), jnp.float32),
            pltpu.VMEM((_DP,), jnp.float32),
            pltpu.SemaphoreType.DMA((8,)),
        ],
        compiler_params=pltpu.CompilerParams(use_tc_tiling_on_sc=False),
    )
    def pool(table_hbm, ids_hbm, b_hbm, out_hbm, idx_v, rows_v, outc_v, b_v, sem):
        wid = lax.axis_index("s") * 2 + lax.axis_index("c")
        pltpu.sync_copy(b_hbm, b_v)
        b_vec = b_v[...]
        pltpu.sync_copy(ids_hbm.at[pl.ds(wid * _RPW * _SEQ, _RPW * _SEQ)], idx_v)

        def gather_desc(ch, buf):
            return pltpu.make_async_copy(
                table_hbm.at[idx_v.at[pl.ds(ch * cl, cl)]], rows_v.at[buf], sem.at[buf]
            )

        for p in range(7):
            gather_desc(p, p).start()

        @pl.loop(0, _NCHUNK // 8)
        def _quad(k):
            for buf in range(8):
                ch = 8 * k + buf
                gather_desc(ch, buf).wait()

                @pl.when(ch + 7 < _NCHUNK)
                def _prefetch():
                    gather_desc(ch + 7, (buf + 7) % 8).start()

                for c in range(_C):
                    base = c * _SEQ

                    def body(j, acc, base=base, buf=buf):
                        j8 = base + 8 * j
                        r = [rows_v[buf, j8 + t] for t in range(8)]
                        s0 = (r[0] + r[1]) + (r[2] + r[3])
                        s1 = (r[4] + r[5]) + (r[6] + r[7])
                        return acc + (s0 + s1)

                    outc_v[c] = lax.fori_loop(0, _SEQ // 8, body, b_vec)
                pltpu.sync_copy(
                    outc_v, out_hbm.at[pl.ds(wid * _RPW + ch * _C, _C)]
                )

    return pool(table, ids_flat, bpad)


def kernel(ids, emb, W, b):
    wp = jnp.zeros((_EMB, _DP), jnp.float32).at[:, :4].set(W.T * (1.0 / _SEQ))
    bpad = jnp.zeros((_DP,), jnp.float32).at[:4].set(b)
    table = _head_table(emb, wp).reshape(_VOCAB, _DP)
    ids_flat = ids.reshape(-1).astype(jnp.int32)
    out = _pooled_head(table, ids_flat, bpad)
    return out[:, :4]


# speedup vs baseline: 1.1650x; 1.0369x over previous
import functools

import jax
import jax.numpy as jnp
from jax import lax
from jax.experimental import pallas as pl
from jax.experimental.pallas import tpu as pltpu
from jax.experimental.pallas import tpu_sc as plsc

_VOCAB = 100000
_EMB = 128
_DP = 16
_BATCH = 4096
_SEQ = 200
_NW = 32
_RPW = _BATCH // _NW
_C = 4
_NCHUNK = _RPW // _C

_MM_BLK = 8192


def _matmul_body(*refs):
    emb_refs, wp_ref, out_ref = refs[:8], refs[8], refs[9]
    wp = wp_ref[...]
    for s in range(8):
        out_ref[:, 16 * s : 16 * (s + 1)] = jnp.dot(
            emb_refs[s][:, 0, 0, :],
            wp,
            preferred_element_type=jnp.float32,
        )


def _head_table(emb, wp):
    emb3 = emb.reshape(_VOCAB // 8, 8, 1, _EMB)
    blk = _MM_BLK // 8
    return pl.pallas_call(
        _matmul_body,
        grid=(pl.cdiv(_VOCAB // 8, blk),),
        in_specs=[
            pl.BlockSpec((blk, 1, 1, _EMB), lambda i, s=s: (i, s, 0, 0))
            for s in range(8)
        ]
        + [pl.BlockSpec((_EMB, _DP), lambda i: (0, 0))],
        out_specs=pl.BlockSpec((blk, _EMB), lambda i: (i, 0)),
        out_shape=jax.ShapeDtypeStruct((_VOCAB // 8, _EMB), jnp.float32),
    )(*([emb3] * 8), wp)


def _pooled_head(table, ids_flat, bpad):
    mesh = plsc.VectorSubcoreMesh(core_axis_name="c", subcore_axis_name="s")

    cl = _C * _SEQ

    @functools.partial(
        pl.kernel,
        out_type=jax.ShapeDtypeStruct((_BATCH, _DP), jnp.float32),
        mesh=mesh,
        scratch_types=[
            pltpu.VMEM((_RPW * _SEQ,), jnp.int32),
            pltpu.VMEM((8, cl, _DP), jnp.float32),
            pltpu.VMEM((_C, _DP), jnp.float32),
            pltpu.VMEM((_DP,), jnp.float32),
            pltpu.SemaphoreType.DMA((8,)),
        ],
        compiler_params=pltpu.CompilerParams(use_tc_tiling_on_sc=False),
    )
    def pool(table_hbm, ids_hbm, b_hbm, out_hbm, idx_v, rows_v, outc_v, b_v, sem):
        wid = lax.axis_index("s") * 2 + lax.axis_index("c")
        pltpu.sync_copy(b_hbm, b_v)
        b_vec = b_v[...]
        pltpu.sync_copy(ids_hbm.at[pl.ds(wid * _RPW * _SEQ, _RPW * _SEQ)], idx_v)

        def gather_desc(ch, buf):
            return pltpu.make_async_copy(
                table_hbm.at[idx_v.at[pl.ds(ch * cl, cl)]], rows_v.at[buf], sem.at[buf]
            )

        for p in range(7):
            gather_desc(p, p).start()

        @pl.loop(0, _NCHUNK // 8)
        def _quad(k):
            for buf in range(8):
                ch = 8 * k + buf
                gather_desc(ch, buf).wait()

                @pl.when(ch + 7 < _NCHUNK)
                def _prefetch():
                    gather_desc(ch + 7, (buf + 7) % 8).start()

                for c in range(_C):
                    base = c * _SEQ

                    def body(j, acc, base=base, buf=buf):
                        j8 = base + 8 * j
                        r = [rows_v[buf, j8 + t] for t in range(8)]
                        s0 = (r[0] + r[1]) + (r[2] + r[3])
                        s1 = (r[4] + r[5]) + (r[6] + r[7])
                        return acc + (s0 + s1)

                    outc_v[c] = lax.fori_loop(0, _SEQ // 8, body, b_vec)
                pltpu.sync_copy(
                    outc_v, out_hbm.at[pl.ds(wid * _RPW + ch * _C, _C)]
                )

    return pool(table, ids_flat, bpad)


def kernel(ids, emb, W, b):
    wp = jnp.zeros((_EMB, _DP), jnp.float32).at[:, :4].set(W.T * (1.0 / _SEQ))
    bpad = jnp.zeros((_DP,), jnp.float32).at[:4].set(b)
    table = _head_table(emb, wp).reshape(_VOCAB, _DP)
    ids_flat = ids.reshape(-1).astype(jnp.int32)
    out = _pooled_head(table, ids_flat, bpad)
    return out[:, :4]


# speedup vs baseline: 1.1928x; 1.0239x over previous
import functools

import jax
import jax.numpy as jnp
from jax import lax
from jax.experimental import pallas as pl
from jax.experimental.pallas import tpu as pltpu
from jax.experimental.pallas import tpu_sc as plsc

_VOCAB = 100000
_EMB = 128
_DP = 16
_BATCH = 4096
_SEQ = 200
_NW = 32
_RPW = _BATCH // _NW
_C = 4
_NCHUNK = _RPW // _C

_MM_BLK = 8192


def _matmul_body(*refs):
    emb_refs, wp_ref, out_ref = refs[:8], refs[8], refs[9]
    wp = wp_ref[...]
    for s in range(8):
        out_ref[:, 16 * s : 16 * (s + 1)] = jnp.dot(
            emb_refs[s][:, 0, 0, :],
            wp,
            preferred_element_type=jnp.float32,
        )


def _head_table(emb, wp):
    emb3 = emb.reshape(_VOCAB // 8, 8, 1, _EMB)
    blk = _MM_BLK // 8
    return pl.pallas_call(
        _matmul_body,
        grid=(pl.cdiv(_VOCAB // 8, blk),),
        in_specs=[
            pl.BlockSpec((blk, 1, 1, _EMB), lambda i, s=s: (i, s, 0, 0))
            for s in range(8)
        ]
        + [pl.BlockSpec((_EMB, _DP), lambda i: (0, 0))],
        out_specs=pl.BlockSpec((blk, _EMB), lambda i: (i, 0)),
        out_shape=jax.ShapeDtypeStruct((_VOCAB // 8, _EMB), jnp.float32),
    )(*([emb3] * 8), wp)


def _pooled_head(table, ids2, bpad):
    mesh = plsc.VectorSubcoreMesh(core_axis_name="c", subcore_axis_name="s")

    @functools.partial(
        pl.kernel,
        out_type=jax.ShapeDtypeStruct((_BATCH, _DP), jnp.float32),
        mesh=mesh,
        scratch_types=[
            pltpu.VMEM((_RPW, _SEQ), jnp.int32),
            pltpu.VMEM((8, _SEQ, _DP), jnp.float32),
            pltpu.VMEM((8, _DP), jnp.float32),
            pltpu.VMEM((_DP,), jnp.float32),
            pltpu.SemaphoreType.DMA((8,)),
        ],
        compiler_params=pltpu.CompilerParams(use_tc_tiling_on_sc=False),
    )
    def pool(table_hbm, ids_hbm, b_hbm, out_hbm, idx_v, rows_v, outc_v, b_v, sem):
        wid = lax.axis_index("s") * 2 + lax.axis_index("c")
        pltpu.sync_copy(b_hbm, b_v)
        b_vec = b_v[...]
        pltpu.sync_copy(ids_hbm.at[pl.ds(wid * _RPW, _RPW), :], idx_v)

        def gather_desc(row, buf):
            return pltpu.make_async_copy(
                table_hbm.at[idx_v.at[row]], rows_v.at[buf], sem.at[buf]
            )

        for p in range(7):
            gather_desc(p, p).start()

        @pl.loop(0, _RPW // 8)
        def _group(k):
            for buf in range(8):
                row = 8 * k + buf
                gather_desc(row, buf).wait()

                @pl.when(row + 7 < _RPW)
                def _prefetch():
                    gather_desc(row + 7, (buf + 7) % 8).start()

                def body(j, acc, buf=buf):
                    j8 = 8 * j
                    r = [rows_v[buf, j8 + t] for t in range(8)]
                    s0 = (r[0] + r[1]) + (r[2] + r[3])
                    s1 = (r[4] + r[5]) + (r[6] + r[7])
                    return acc + (s0 + s1)

                outc_v[buf] = lax.fori_loop(0, _SEQ // 8, body, b_vec)
            pltpu.sync_copy(outc_v, out_hbm.at[pl.ds(wid * _RPW + 8 * k, 8)])

    return pool(table, ids2, bpad)


def kernel(ids, emb, W, b):
    wp = jnp.zeros((_EMB, _DP), jnp.float32).at[:, :4].set(W.T * (1.0 / _SEQ))
    bpad = jnp.zeros((_DP,), jnp.float32).at[:4].set(b)
    table = _head_table(emb, wp).reshape(_VOCAB, _DP)
    out = _pooled_head(table, ids.astype(jnp.int32), bpad)
    return out[:, :4]
